# baseline (speedup 1.0000x reference)
import functools

import jax
import jax.numpy as jnp
from jax import lax
from jax.experimental import pallas as pl
from jax.experimental.pallas import tpu as pltpu
from jax.experimental.pallas import tpu_sc as plsc

GRID = (360, 360)
NUM_BATCH = 4
POOL_DIM = 256
FEA_COMPRE = 32
NUM_VOX = NUM_BATCH * GRID[0] * GRID[1]
N_PTS = 262144

NW = 32
LIST_RANGE = 16384
PASS_VOX = 8192
SUB_VOX = 256
NUM_PASS = 64
BATCH = 128
CHUNK = 4096
FLUSH = 2048
LIST_CAP = N_PTS + FLUSH


def _vfe_body(fea_ref, w_ref, b_ref, out_ref):
    out_ref[...] = jax.nn.relu(
        jnp.dot(fea_ref[...], w_ref[...], preferred_element_type=jnp.float32)
        + b_ref[...]
    )


def _vfe_matmul(pt_fea, W_vfe, b_vfe):
    n = pt_fea.shape[0]
    blk = 2048
    return pl.pallas_call(
        _vfe_body,
        grid=(n // blk,),
        in_specs=[
            pl.BlockSpec((blk, pt_fea.shape[1]), lambda i: (i, 0)),
            pl.BlockSpec((pt_fea.shape[1], POOL_DIM), lambda i: (0, 0)),
            pl.BlockSpec((POOL_DIM,), lambda i: (0,)),
        ],
        out_specs=pl.BlockSpec((blk, POOL_DIM), lambda i: (i, 0)),
        out_shape=jax.ShapeDtypeStruct((n, POOL_DIM), jnp.float32),
    )(pt_fea, W_vfe, b_vfe)


def _comp_body(pool_ref, w_ref, b_ref, out_ref):
    out_ref[...] = jax.nn.relu(
        jnp.dot(pool_ref[...], w_ref[...], preferred_element_type=jnp.float32)
        + b_ref[...]
    )


def _comp_matmul(dense, W_comp, b_comp):
    blk = 2880
    grid = NUM_VOX // blk
    return pl.pallas_call(
        _comp_body,
        grid=(grid,),
        in_specs=[
            pl.BlockSpec((blk, POOL_DIM), lambda i: (i, 0)),
            pl.BlockSpec((POOL_DIM, FEA_COMPRE), lambda i: (0, 0)),
            pl.BlockSpec((FEA_COMPRE,), lambda i: (0,)),
        ],
        out_specs=pl.BlockSpec((blk, FEA_COMPRE), lambda i: (i, 0)),
        out_shape=jax.ShapeDtypeStruct((NUM_VOX, FEA_COMPRE), jnp.float32),
    )(dense, W_comp, b_comp)


def _append_compact(ref, off, x, m):
    mi = m.astype(jnp.int32)
    pos = off + plsc.cumsum(mi) - mi
    plsc.store_scatter(ref, [pos], x, mask=m)


def _mesh():
    return plsc.VectorSubcoreMesh(core_axis_name="c", subcore_axis_name="s")


def _bucket_kernel(keys):
    def body(keys_hbm, lkeys_hbm, lidx_hbm, counts_hbm,
             kchunk, kb, ib, cntv, sem):
        wid = lax.axis_index("s") * 2 + lax.axis_index("c")
        iota = lax.iota(jnp.int32, 16)
        lo = wid * LIST_RANGE
        hi = lo + LIST_RANGE

        def chunk_body(ci, carry):
            off, goff = carry
            pltpu.sync_copy(keys_hbm.at[pl.ds(ci * CHUNK, CHUNK)], kchunk)

            def vec_body(i, carry):
                off, goff = carry
                k = kchunk[pl.ds(i * 16, 16)]
                m = (k >= lo) & (k < hi)
                idxv = ci * CHUNK + i * 16 + iota
                _append_compact(kb, off, k, m)
                _append_compact(ib, off, idxv, m)
                off = off + jnp.sum(m.astype(jnp.int32), axis=0)

                def do_flush(carry):
                    off, goff = carry
                    base = pl.multiple_of(wid * LIST_CAP + goff, FLUSH)
                    pltpu.sync_copy(kb.at[pl.ds(0, FLUSH)],
                                    lkeys_hbm.at[pl.ds(base, FLUSH)])
                    pltpu.sync_copy(ib.at[pl.ds(0, FLUSH)],
                                    lidx_hbm.at[pl.ds(base, FLUSH)])
                    kb[pl.ds(0, 16)] = kb[pl.ds(FLUSH, 16)]
                    ib[pl.ds(0, 16)] = ib[pl.ds(FLUSH, 16)]
                    return off - FLUSH, goff + FLUSH

                return lax.cond(off >= FLUSH, do_flush, lambda c: c,
                                (off, goff))

            return lax.fori_loop(0, CHUNK // 16, vec_body, (off, goff))

        off, goff = lax.fori_loop(0, N_PTS // CHUNK, chunk_body, (0, 0))
        base = pl.multiple_of(wid * LIST_CAP + goff, FLUSH)
        pltpu.sync_copy(kb.at[pl.ds(0, FLUSH)],
                        lkeys_hbm.at[pl.ds(base, FLUSH)])
        pltpu.sync_copy(ib.at[pl.ds(0, FLUSH)],
                        lidx_hbm.at[pl.ds(base, FLUSH)])
        cntv[...] = jnp.broadcast_to(goff + off, (16,)).astype(jnp.int32)
        pltpu.sync_copy(cntv, counts_hbm.at[pl.ds(wid * 16, 16)])

    f = pl.kernel(
        body,
        out_type=[
            jax.ShapeDtypeStruct((NW * LIST_CAP,), jnp.int32),
            jax.ShapeDtypeStruct((NW * LIST_CAP,), jnp.int32),
            jax.ShapeDtypeStruct((NW * 16,), jnp.int32),
        ],
        mesh=_mesh(),
        compiler_params=pltpu.CompilerParams(needs_layout_passes=False),
        scratch_types=[
            pltpu.VMEM((CHUNK,), jnp.int32),
            pltpu.VMEM((FLUSH + 16,), jnp.int32),
            pltpu.VMEM((FLUSH + 16,), jnp.int32),
            pltpu.VMEM((16,), jnp.int32),
            pltpu.SemaphoreType.DMA,
        ],
    )
    return f(keys)


def _scatter_max_kernel(lkeys, lidx, counts, processed):
    def body(lkeys_hbm, lidx_hbm, counts_hbm, proc_hbm, dense_hbm,
             kchunk, jchunk, vb, jb, idxb, rows, acc, cntv, sem, gsem):
        wid = lax.axis_index("s") * 2 + lax.axis_index("c")
        iota = lax.iota(jnp.int32, 16)
        zero16 = jnp.zeros((16,), jnp.float32)

        def process_batch(_):
            for j in range(BATCH // 16):
                idxb[pl.ds(j * 16, 16)] = jb[pl.ds(j * 16, 16)]
            pltpu.async_copy(proc_hbm.at[idxb], rows, gsem).wait()

            def pt(k, _):
                kv = vb[pl.ds((k // 16) * 16, 16)]
                v = jnp.sum(jnp.where(iota == (k % 16), kv, 0), axis=0)
                for j in range(POOL_DIM // 16):
                    sl = pl.ds(j * 16, 16)
                    acc[v, sl] = jnp.maximum(acc[v, sl], rows[k, sl])
                return 0

            lax.fori_loop(0, BATCH, pt, 0)
            vb[pl.ds(0, 16)] = vb[pl.ds(BATCH, 16)]
            jb[pl.ds(0, 16)] = jb[pl.ds(BATCH, 16)]
            return ()

        def pass_body(p, _):
            lo = p * PASS_VOX + wid * SUB_VOX
            s = p // 2

            def zrow(v, _):
                for j in range(POOL_DIM // 16):
                    acc[v, pl.ds(j * 16, 16)] = zero16
                return 0

            lax.fori_loop(0, SUB_VOX + 1, zrow, 0)

            pltpu.sync_copy(counts_hbm.at[pl.ds(pl.multiple_of(s * 16, 16), 16)], cntv)
            cnt = jnp.max(cntv[...], axis=0)

            def chunk_body(ci, off):
                cbase = pl.multiple_of(s * LIST_CAP + ci * CHUNK, CHUNK)
                pltpu.sync_copy(lkeys_hbm.at[pl.ds(cbase, CHUNK)], kchunk)
                pltpu.sync_copy(lidx_hbm.at[pl.ds(cbase, CHUNK)], jchunk)

                def vec_body(i, off):
                    k = kchunk[pl.ds(i * 16, 16)]
                    jx = jchunk[pl.ds(i * 16, 16)]
                    valid = (ci * CHUNK + i * 16 + iota) < cnt
                    m = valid & (k >= lo) & (k < lo + SUB_VOX)
                    _append_compact(vb, off, k - lo, m)
                    _append_compact(jb, off, jx, m)
                    off = off + jnp.sum(m.astype(jnp.int32), axis=0)

                    def flush_batch(off):
                        process_batch(())
                        return off - BATCH

                    return lax.cond(off >= BATCH, flush_batch,
                                    lambda o: o, off)

                nvec = jnp.minimum(CHUNK, cnt - ci * CHUNK)
                nvec = (nvec + 15) // 16
                return lax.fori_loop(0, nvec, vec_body, off)

            nch = (cnt + CHUNK - 1) // CHUNK
            off = lax.fori_loop(0, nch, chunk_body, 0)

            def drain(off):
                for j in range(BATCH // 16):
                    sl = pl.ds(j * 16, 16)
                    lanepos = j * 16 + iota
                    vb[sl] = jnp.where(lanepos < off, vb[sl], SUB_VOX)
                    jb[sl] = jnp.where(lanepos < off, jb[sl], 0)
                process_batch(())
                return 0

            lax.cond(off > 0, drain, lambda o: 0, off)

            pltpu.sync_copy(acc.at[pl.ds(0, SUB_VOX)],
                            dense_hbm.at[pl.ds(pl.multiple_of(lo, SUB_VOX), SUB_VOX)])
            return ()

        lax.fori_loop(0, NUM_PASS, pass_body, ())

    f = pl.kernel(
        body,
        out_type=jax.ShapeDtypeStruct((NUM_PASS * PASS_VOX, POOL_DIM),
                                      jnp.float32),
        mesh=_mesh(),
        compiler_params=pltpu.CompilerParams(needs_layout_passes=False),
        scratch_types=[
            pltpu.VMEM((CHUNK,), jnp.int32),
            pltpu.VMEM((CHUNK,), jnp.int32),
            pltpu.VMEM((BATCH + 16,), jnp.int32),
            pltpu.VMEM((BATCH + 16,), jnp.int32),
            pltpu.VMEM((BATCH,), jnp.int32),
            pltpu.VMEM((BATCH, POOL_DIM), jnp.float32),
            pltpu.VMEM((SUB_VOX + 1, POOL_DIM), jnp.float32),
            pltpu.VMEM((16,), jnp.int32),
            pltpu.SemaphoreType.DMA,
            pltpu.SemaphoreType.DMA,
        ],
    )
    return f(lkeys, lidx, counts, processed)


def kernel(pt_fea, grid_ind, batch_ids, W_vfe, b_vfe, W_comp, b_comp):
    keys = (batch_ids * (GRID[0] * GRID[1])
            + grid_ind[:, 0] * GRID[1] + grid_ind[:, 1]).astype(jnp.int32)
    processed = _vfe_matmul(pt_fea, W_vfe, b_vfe)
    lkeys, lidx, counts = _bucket_kernel(keys)
    dense = _scatter_max_kernel(lkeys, lidx, counts, processed)
    compressed = _comp_matmul(dense, W_comp, b_comp)
    out = compressed.reshape(NUM_BATCH, GRID[0], GRID[1], FEA_COMPRE)
    return jnp.transpose(out, (0, 3, 1, 2))
